# baseline (speedup 1.0000x reference)
import functools

import jax
import jax.numpy as jnp
from jax.experimental import pallas as pl
from jax.experimental.pallas import tpu as pltpu

WIN = 8
KEEP_RATIO = 0.5
MIN_KEEP = 1

_ANY = pl.ANY


def _pool_mats(H, W, NH, NW, dtype=jnp.float32):
    r = jax.lax.broadcasted_iota(jnp.int32, (NH, H), 0)
    c = jax.lax.broadcasted_iota(jnp.int32, (NH, H), 1)
    eh = (c // WIN == r).astype(dtype)
    r2 = jax.lax.broadcasted_iota(jnp.int32, (W, NW), 0)
    c2 = jax.lax.broadcasted_iota(jnp.int32, (W, NW), 1)
    ew = (r2 // WIN == c2).astype(dtype)
    return eh, ew


def _topk_window_mask(pooled, keep, NH, NW):
    N = NH * NW
    pi = jax.lax.bitcast_convert_type(pooled, jnp.int32)

    def cnt_ge(t):
        return jnp.sum((pi >= t).astype(jnp.int32), keepdims=True).reshape(1, 1)

    def body(_, lohi):
        lo, hi = lohi
        mid = lo + ((hi - lo) // 2) + ((hi - lo) & 1)
        ok = cnt_ge(mid) >= keep
        return (jnp.where(ok, mid, lo), jnp.where(ok, hi, mid - 1))

    lo0 = jnp.zeros((1, 1), jnp.int32)
    hi0 = jnp.full((1, 1), 0x7F800000, jnp.int32)
    vstar, _ = jax.lax.fori_loop(0, 31, body, (lo0, hi0))

    gt = pi > vstar
    eq = pi == vstar
    need = keep - jnp.sum(gt.astype(jnp.int32), keepdims=True).reshape(1, 1)

    flat = (jax.lax.broadcasted_iota(jnp.int32, (NH, NW), 0) * NW
            + jax.lax.broadcasted_iota(jnp.int32, (NH, NW), 1))

    def cnt_eq_lt(m):
        return jnp.sum((eq & (flat < m)).astype(jnp.int32),
                       keepdims=True).reshape(1, 1)

    def body2(_, lohi):
        lo, hi = lohi
        mid = (lo + hi) // 2
        ok = cnt_eq_lt(mid) >= need
        return (jnp.where(ok, lo, mid + 1), jnp.where(ok, mid, hi))

    lo0 = jnp.full((1, 1), 1, jnp.int32)
    hi0 = jnp.full((1, 1), N, jnp.int32)
    mcut, _ = jax.lax.fori_loop(0, 10, body2, (lo0, hi0))

    return (gt | (eq & (flat < mcut))).astype(jnp.float32)


def _fused_kernel(x_hbm, o_hbm, wm_ref, tm_ref,
                  buf, isem, osem, *, BT, C, H, W, NH, NW, keep, ns, R, sch):
    bt = pl.program_id(0)
    total = BT * ns

    def slab_in(g, frame, cidx):
        slot = jax.lax.rem(g, R)
        return pltpu.make_async_copy(
            x_hbm.at[frame, pl.ds(cidx * sch, sch)], buf.at[slot],
            isem.at[slot])

    def slab_out(g, frame, cidx):
        slot = jax.lax.rem(g, R)
        return pltpu.make_async_copy(
            buf.at[slot], o_hbm.at[frame, pl.ds(cidx * sch, sch)],
            osem.at[slot])

    @pl.when(bt == 0)
    def _():
        for g in range(min(R, total)):
            slab_in(jnp.int32(g), jnp.int32(g // ns), jnp.int32(g % ns)).start()

    e = None
    for s in range(ns):
        g = bt * ns + s
        slab_in(g, bt, jnp.int32(s)).wait()
        slab = buf[jax.lax.rem(g, R)]
        for k in range(0, sch, 16):
            part = jnp.sum(jnp.abs(slab[k:k + 16]), axis=0)
            e = part if e is None else e + part
    energy = e / jnp.float32(C)

    eh, ew = _pool_mats(H, W, NH, NW)
    wsum = jax.lax.dot_general(
        energy, ew, (((1,), (0,)), ((), ())),
        precision=jax.lax.Precision.HIGHEST,
        preferred_element_type=jnp.float32)
    hsum = jax.lax.dot_general(
        eh, wsum, (((1,), (0,)), ((), ())),
        precision=jax.lax.Precision.HIGHEST,
        preferred_element_type=jnp.float32)
    pooled = hsum / jnp.float32(WIN * WIN)

    wmask = _topk_window_mask(pooled, keep, NH, NW)
    wm_ref[0] = wmask
    t1 = jax.lax.dot_general(
        eh, wmask, (((0,), (0,)), ((), ())),
        preferred_element_type=jnp.float32)
    tmask = jax.lax.dot_general(
        t1, ew, (((1,), (1,)), ((), ())),
        preferred_element_type=jnp.float32)
    tm_ref[0] = tmask

    for s in range(ns):
        g = bt * ns + s
        slot = jax.lax.rem(g, R)
        buf[slot] = buf[slot] * tmask[None, :, :]
        slab_out(g, bt, jnp.int32(s)).start()

    for s in range(ns):
        g = bt * ns + s
        slab_out(g, bt, jnp.int32(s)).wait()
        g2 = g + R

        @pl.when(g2 < total)
        def _(g2=g2):
            slab_in(g2, g2 // ns, jax.lax.rem(g2, ns)).start()


def kernel(x):
    B, T, C, H, W = x.shape
    assert H % WIN == 0 and W % WIN == 0 and C % 16 == 0
    NH, NW = H // WIN, W // WIN
    N = NH * NW
    keep = min(max(MIN_KEEP, int(N * KEEP_RATIO)), N)
    BT = B * T
    x4 = x.reshape(BT, C, H, W)

    sch = 32 if C % 32 == 0 else 16
    ns = C // sch
    R = min(ns + 2, BT * ns)
    out, wm, tm = pl.pallas_call(
        functools.partial(_fused_kernel, BT=BT, C=C, H=H, W=W,
                          NH=NH, NW=NW, keep=keep, ns=ns, R=R, sch=sch),
        grid=(BT,),
        in_specs=[pl.BlockSpec(memory_space=_ANY)],
        out_specs=[
            pl.BlockSpec(memory_space=_ANY),
            pl.BlockSpec((1, NH, NW), lambda bt: (bt, 0, 0)),
            pl.BlockSpec((1, H, W), lambda bt: (bt, 0, 0)),
        ],
        out_shape=[
            jax.ShapeDtypeStruct((BT, C, H, W), jnp.float32),
            jax.ShapeDtypeStruct((BT, NH, NW), jnp.float32),
            jax.ShapeDtypeStruct((BT, H, W), jnp.float32),
        ],
        scratch_shapes=[
            pltpu.VMEM((R, sch, H, W), jnp.float32),
            pltpu.SemaphoreType.DMA((R,)),
            pltpu.SemaphoreType.DMA((R,)),
        ],
        compiler_params=pltpu.CompilerParams(
            dimension_semantics=("arbitrary",),
            vmem_limit_bytes=62 * 1024 * 1024),
    )(x4)

    pruned = out.reshape(B, T, C, H, W)
    token_mask = tm.reshape(B, T, H, W).astype(jnp.bool_)
    window_mask = wm.reshape(B, T, NH, NW).astype(jnp.bool_)
    return (pruned, token_mask, window_mask)


# speedup vs baseline: 1.1424x; 1.1424x over previous
import functools

import jax
import jax.numpy as jnp
from jax.experimental import pallas as pl
from jax.experimental.pallas import tpu as pltpu

WIN = 8
KEEP_RATIO = 0.5
MIN_KEEP = 1

_ANY = pl.ANY


def _pool_mats(H, W, NH, NW, dtype=jnp.float32):
    r = jax.lax.broadcasted_iota(jnp.int32, (NH, H), 0)
    c = jax.lax.broadcasted_iota(jnp.int32, (NH, H), 1)
    eh = (c // WIN == r).astype(dtype)
    r2 = jax.lax.broadcasted_iota(jnp.int32, (W, NW), 0)
    c2 = jax.lax.broadcasted_iota(jnp.int32, (W, NW), 1)
    ew = (r2 // WIN == c2).astype(dtype)
    return eh, ew


def _topk_window_mask(pooled, keep, NH, NW):
    N = NH * NW
    pi = jax.lax.bitcast_convert_type(pooled, jnp.int32)

    def cnt_ge(t):
        return jnp.sum((pi >= t).astype(jnp.int32), keepdims=True).reshape(1, 1)

    def body(_, lohi):
        lo, hi = lohi
        mid = lo + ((hi - lo) // 2) + ((hi - lo) & 1)
        ok = cnt_ge(mid) >= keep
        return (jnp.where(ok, mid, lo), jnp.where(ok, hi, mid - 1))

    lo0 = jnp.zeros((1, 1), jnp.int32)
    hi0 = jnp.full((1, 1), 0x7F800000, jnp.int32)
    vstar, _ = jax.lax.fori_loop(0, 31, body, (lo0, hi0))

    gt = pi > vstar
    eq = pi == vstar
    need = keep - jnp.sum(gt.astype(jnp.int32), keepdims=True).reshape(1, 1)

    flat = (jax.lax.broadcasted_iota(jnp.int32, (NH, NW), 0) * NW
            + jax.lax.broadcasted_iota(jnp.int32, (NH, NW), 1))

    def cnt_eq_lt(m):
        return jnp.sum((eq & (flat < m)).astype(jnp.int32),
                       keepdims=True).reshape(1, 1)

    def body2(_, lohi):
        lo, hi = lohi
        mid = (lo + hi) // 2
        ok = cnt_eq_lt(mid) >= need
        return (jnp.where(ok, lo, mid + 1), jnp.where(ok, mid, hi))

    lo0 = jnp.full((1, 1), 1, jnp.int32)
    hi0 = jnp.full((1, 1), N, jnp.int32)
    mcut, _ = jax.lax.fori_loop(0, 10, body2, (lo0, hi0))

    return (gt | (eq & (flat < mcut))).astype(jnp.float32)


def _fused_kernel(x_hbm, o_hbm, wm_ref, tm_ref,
                  buf, eacc, isem, osem,
                  *, BT, C, H, W, NH, NW, keep, ns, R, sch):
    bt = pl.program_id(0)
    total = BT * ns

    def slab_in(g, frame, cidx):
        slot = jax.lax.rem(g, R)
        return pltpu.make_async_copy(
            x_hbm.at[frame, pl.ds(cidx * sch, sch)], buf.at[slot],
            isem.at[slot])

    def slab_out(g, frame, cidx):
        slot = jax.lax.rem(g, R)
        return pltpu.make_async_copy(
            buf.at[slot], o_hbm.at[frame, pl.ds(cidx * sch, sch)],
            osem.at[slot])

    def accumulate_energy(frame):
        for s in range(ns):
            g = frame * ns + s
            slab_in(g, frame, jnp.int32(s)).wait()
            slab = buf[jax.lax.rem(g, R)]
            for k in range(0, sch, 16):
                part = jnp.sum(jnp.abs(slab[k:k + 16]), axis=0)
                if s == 0 and k == 0:
                    eacc[...] = part
                else:
                    eacc[...] += part

    @pl.when(bt == 0)
    def _():
        for g in range(min(R, total)):
            slab_in(jnp.int32(g), jnp.int32(g // ns), jnp.int32(g % ns)).start()
        accumulate_energy(bt)

    energy = eacc[...] / jnp.float32(C)

    eh, ew = _pool_mats(H, W, NH, NW)
    wsum = jax.lax.dot_general(
        energy, ew, (((1,), (0,)), ((), ())),
        precision=jax.lax.Precision.HIGHEST,
        preferred_element_type=jnp.float32)
    hsum = jax.lax.dot_general(
        eh, wsum, (((1,), (0,)), ((), ())),
        precision=jax.lax.Precision.HIGHEST,
        preferred_element_type=jnp.float32)
    pooled = hsum / jnp.float32(WIN * WIN)

    wmask = _topk_window_mask(pooled, keep, NH, NW)
    wm_ref[0] = wmask
    t1 = jax.lax.dot_general(
        eh, wmask, (((0,), (0,)), ((), ())),
        preferred_element_type=jnp.float32)
    tmask = jax.lax.dot_general(
        t1, ew, (((1,), (1,)), ((), ())),
        preferred_element_type=jnp.float32)
    tm_ref[0] = tmask

    for s in range(ns):
        g = bt * ns + s
        slot = jax.lax.rem(g, R)
        buf[slot] = buf[slot] * tmask[None, :, :]
        slab_out(g, bt, jnp.int32(s)).start()

    for s in range(ns):
        g = bt * ns + s
        slab_out(g, bt, jnp.int32(s)).wait()
        g2 = g + R

        @pl.when(g2 < total)
        def _(g2=g2):
            slab_in(g2, g2 // ns, jax.lax.rem(g2, ns)).start()

        @pl.when(bt + 1 < BT)
        def _(s=s):
            gn = (bt + 1) * ns + s
            slab_in(gn, bt + 1, jnp.int32(s)).wait()
            slab = buf[jax.lax.rem(gn, R)]
            for k in range(0, sch, 16):
                part = jnp.sum(jnp.abs(slab[k:k + 16]), axis=0)
                if s == 0 and k == 0:
                    eacc[...] = part
                else:
                    eacc[...] += part


def kernel(x):
    B, T, C, H, W = x.shape
    assert H % WIN == 0 and W % WIN == 0 and C % 16 == 0
    NH, NW = H // WIN, W // WIN
    N = NH * NW
    keep = min(max(MIN_KEEP, int(N * KEEP_RATIO)), N)
    BT = B * T
    x4 = x.reshape(BT, C, H, W)

    sch = 16
    ns = C // sch
    R = min(ns + 4, BT * ns)
    out, wm, tm = pl.pallas_call(
        functools.partial(_fused_kernel, BT=BT, C=C, H=H, W=W,
                          NH=NH, NW=NW, keep=keep, ns=ns, R=R, sch=sch),
        grid=(BT,),
        in_specs=[pl.BlockSpec(memory_space=_ANY)],
        out_specs=[
            pl.BlockSpec(memory_space=_ANY),
            pl.BlockSpec((1, NH, NW), lambda bt: (bt, 0, 0)),
            pl.BlockSpec((1, H, W), lambda bt: (bt, 0, 0)),
        ],
        out_shape=[
            jax.ShapeDtypeStruct((BT, C, H, W), jnp.float32),
            jax.ShapeDtypeStruct((BT, NH, NW), jnp.float32),
            jax.ShapeDtypeStruct((BT, H, W), jnp.float32),
        ],
        scratch_shapes=[
            pltpu.VMEM((R, sch, H, W), jnp.float32),
            pltpu.VMEM((H, W), jnp.float32),
            pltpu.SemaphoreType.DMA((R,)),
            pltpu.SemaphoreType.DMA((R,)),
        ],
        compiler_params=pltpu.CompilerParams(
            dimension_semantics=("arbitrary",),
            vmem_limit_bytes=62 * 1024 * 1024),
    )(x4)

    pruned = out.reshape(B, T, C, H, W)
    token_mask = tm.reshape(B, T, H, W).astype(jnp.bool_)
    window_mask = wm.reshape(B, T, NH, NW).astype(jnp.bool_)
    return (pruned, token_mask, window_mask)


# speedup vs baseline: 1.1729x; 1.0266x over previous
import functools

import jax
import jax.numpy as jnp
from jax.experimental import pallas as pl
from jax.experimental.pallas import tpu as pltpu

WIN = 8
KEEP_RATIO = 0.5
MIN_KEEP = 1

_ANY = pl.ANY


def _pool_mats(H, W, NH, NW, dtype=jnp.float32):
    r = jax.lax.broadcasted_iota(jnp.int32, (NH, H), 0)
    c = jax.lax.broadcasted_iota(jnp.int32, (NH, H), 1)
    eh = (c // WIN == r).astype(dtype)
    r2 = jax.lax.broadcasted_iota(jnp.int32, (W, NW), 0)
    c2 = jax.lax.broadcasted_iota(jnp.int32, (W, NW), 1)
    ew = (r2 // WIN == c2).astype(dtype)
    return eh, ew


def _topk_window_mask(pooled, keep, NH, NW):
    N = NH * NW
    pi = jax.lax.bitcast_convert_type(pooled, jnp.int32)

    def cnt_ge(t):
        return jnp.sum((pi >= t).astype(jnp.int32), keepdims=True).reshape(1, 1)

    def body(_, lohi):
        lo, hi = lohi
        mid = lo + ((hi - lo) // 2) + ((hi - lo) & 1)
        ok = cnt_ge(mid) >= keep
        return (jnp.where(ok, mid, lo), jnp.where(ok, hi, mid - 1))

    lo0 = jnp.zeros((1, 1), jnp.int32)
    hi0 = jnp.full((1, 1), 0x7F800000, jnp.int32)
    vstar, _ = jax.lax.fori_loop(0, 31, body, (lo0, hi0))

    gt = pi > vstar
    eq = pi == vstar
    need = keep - jnp.sum(gt.astype(jnp.int32), keepdims=True).reshape(1, 1)

    flat = (jax.lax.broadcasted_iota(jnp.int32, (NH, NW), 0) * NW
            + jax.lax.broadcasted_iota(jnp.int32, (NH, NW), 1))

    def cnt_eq_lt(m):
        return jnp.sum((eq & (flat < m)).astype(jnp.int32),
                       keepdims=True).reshape(1, 1)

    def body2(_, lohi):
        lo, hi = lohi
        mid = (lo + hi) // 2
        ok = cnt_eq_lt(mid) >= need
        return (jnp.where(ok, lo, mid + 1), jnp.where(ok, mid, hi))

    lo0 = jnp.full((1, 1), 1, jnp.int32)
    hi0 = jnp.full((1, 1), N, jnp.int32)
    mcut, _ = jax.lax.fori_loop(0, 10, body2, (lo0, hi0))

    return (gt | (eq & (flat < mcut))).astype(jnp.float32)


def _fused_kernel(x_hbm, o_hbm, wm_ref, tm_ref,
                  buf, eacc, isem, osem,
                  *, BT, C, H, W, NH, NW, keep, ns, R, sch):
    bt = pl.program_id(0)
    total = BT * ns

    def slab_in(g, frame, cidx):
        slot = jax.lax.rem(g, R)
        return pltpu.make_async_copy(
            x_hbm.at[frame, pl.ds(cidx * sch, sch)], buf.at[slot],
            isem.at[slot])

    def slab_out(g, frame, cidx):
        slot = jax.lax.rem(g, R)
        return pltpu.make_async_copy(
            buf.at[slot], o_hbm.at[frame, pl.ds(cidx * sch, sch)],
            osem.at[slot])

    def accumulate_energy(frame):
        for s in range(ns):
            g = frame * ns + s
            slab_in(g, frame, jnp.int32(s)).wait()
            slab = buf[jax.lax.rem(g, R)]
            for k in range(0, sch, 16):
                part = jnp.sum(jnp.abs(slab[k:k + 16]), axis=0)
                if s == 0 and k == 0:
                    eacc[...] = part
                else:
                    eacc[...] += part

    @pl.when(bt == 0)
    def _():
        for g in range(min(R, total)):
            slab_in(jnp.int32(g), jnp.int32(g // ns), jnp.int32(g % ns)).start()
        accumulate_energy(bt)

    energy = eacc[...] / jnp.float32(C)

    eh, ew = _pool_mats(H, W, NH, NW)
    wsum = jax.lax.dot_general(
        energy, ew, (((1,), (0,)), ((), ())),
        precision=jax.lax.Precision.HIGHEST,
        preferred_element_type=jnp.float32)
    hsum = jax.lax.dot_general(
        eh, wsum, (((1,), (0,)), ((), ())),
        precision=jax.lax.Precision.HIGHEST,
        preferred_element_type=jnp.float32)
    pooled = hsum / jnp.float32(WIN * WIN)

    wmask = _topk_window_mask(pooled, keep, NH, NW)
    wm_ref[0] = wmask
    t1 = jax.lax.dot_general(
        eh, wmask, (((0,), (0,)), ((), ())),
        preferred_element_type=jnp.float32)
    tmask = jax.lax.dot_general(
        t1, ew, (((1,), (1,)), ((), ())),
        preferred_element_type=jnp.float32)
    tm_ref[0] = tmask

    for s in range(ns):
        g = bt * ns + s
        slot = jax.lax.rem(g, R)
        buf[slot] = buf[slot] * tmask[None, :, :]
        slab_out(g, bt, jnp.int32(s)).start()

    for s in range(ns):
        g = bt * ns + s
        slab_out(g, bt, jnp.int32(s)).wait()
        g2 = g + R

        @pl.when(g2 < total)
        def _(g2=g2):
            slab_in(g2, g2 // ns, jax.lax.rem(g2, ns)).start()

        @pl.when(bt + 1 < BT)
        def _(s=s):
            gn = (bt + 1) * ns + s
            slab_in(gn, bt + 1, jnp.int32(s)).wait()
            slab = buf[jax.lax.rem(gn, R)]
            for k in range(0, sch, 16):
                part = jnp.sum(jnp.abs(slab[k:k + 16]), axis=0)
                if s == 0 and k == 0:
                    eacc[...] = part
                else:
                    eacc[...] += part


def kernel(x):
    B, T, C, H, W = x.shape
    assert H % WIN == 0 and W % WIN == 0 and C % 16 == 0
    NH, NW = H // WIN, W // WIN
    N = NH * NW
    keep = min(max(MIN_KEEP, int(N * KEEP_RATIO)), N)
    BT = B * T
    x4 = x.reshape(BT, C, H, W)

    sch = 16
    ns = C // sch
    R = min(ns + 5, BT * ns)
    out, wm, tm = pl.pallas_call(
        functools.partial(_fused_kernel, BT=BT, C=C, H=H, W=W,
                          NH=NH, NW=NW, keep=keep, ns=ns, R=R, sch=sch),
        grid=(BT,),
        in_specs=[pl.BlockSpec(memory_space=_ANY)],
        out_specs=[
            pl.BlockSpec(memory_space=_ANY),
            pl.BlockSpec((1, NH, NW), lambda bt: (bt, 0, 0)),
            pl.BlockSpec((1, H, W), lambda bt: (bt, 0, 0)),
        ],
        out_shape=[
            jax.ShapeDtypeStruct((BT, C, H, W), jnp.float32),
            jax.ShapeDtypeStruct((BT, NH, NW), jnp.float32),
            jax.ShapeDtypeStruct((BT, H, W), jnp.float32),
        ],
        scratch_shapes=[
            pltpu.VMEM((R, sch, H, W), jnp.float32),
            pltpu.VMEM((H, W), jnp.float32),
            pltpu.SemaphoreType.DMA((R,)),
            pltpu.SemaphoreType.DMA((R,)),
        ],
        compiler_params=pltpu.CompilerParams(
            dimension_semantics=("arbitrary",),
            vmem_limit_bytes=63 * 1024 * 1024),
    )(x4)

    pruned = out.reshape(B, T, C, H, W)
    token_mask = tm.reshape(B, T, H, W).astype(jnp.bool_)
    window_mask = wm.reshape(B, T, NH, NW).astype(jnp.bool_)
    return (pruned, token_mask, window_mask)
